# baseline (speedup 1.0000x reference)
import functools

import jax
import jax.numpy as jnp
from jax import lax
from jax.experimental import pallas as pl
from jax.experimental.pallas import tpu as pltpu
from jax.experimental.pallas import tpu_sc as plsc

BATCH = 4096
MAX_LEN = 200
EMBED = 64
LP = 224
HALF = LP // 4
VOCAB2 = 500000
NC, NS, LANES = 2, 16, 16
NW = NC * NS
BPW = BATCH // NW
UPW = 4 * BPW
NCH = EMBED // LANES
NB = 4
STW = 112


def _encoder(xf_hbm, xs_hbm, t128_hbm, f_hbm, out_hbm,
             idx_v, idx2_v, f_v, buf0, buf1, buf2, buf3, out_v,
             sem0, sem1, sem2, sem3):
    bufs = (buf0, buf1, buf2, buf3)
    sems = (sem0, sem1, sem2, sem3)
    wid = lax.axis_index("s") * NC + lax.axis_index("c")
    base = wid * BPW

    pltpu.sync_copy(xf_hbm.at[pl.ds(base * LP, BPW * LP)],
                    idx_v.at[pl.ds(0, BPW * LP)])
    pltpu.sync_copy(xs_hbm.at[pl.ds(wid * UPW, UPW)], idx2_v)
    pltpu.sync_copy(f_hbm, f_v)

    def fire(u, buf, sem):
        pltpu.make_async_copy(t128_hbm.at[idx2_v.at[u]], buf, sem).start()

    def drain(buf, sem):
        pltpu.make_async_copy(t128_hbm.at[idx2_v.at[0]], buf, sem).wait()

    def accumulate(u, j, buf, acc):
        fbase = j * HALF
        pu = u * HALF

        def body(rr, acc):
            r0 = rr * 8
            iv_vec = idx_v[pl.ds(pu + r0, 16)]
            for i in range(8):
                r = r0 + i
                spl = iv_vec[jnp.full((LANES,), i, jnp.int32)]
                w = (spl & 1).astype(jnp.float32)
                acc = tuple(
                    acc[c] + (
                        lo + (hi - lo) * w
                    ) * f_v[fbase + r, pl.ds(c * LANES, LANES)]
                    for c, (lo, hi) in enumerate(
                        (buf[r, pl.ds(c * LANES, LANES)],
                         buf[r, pl.ds(EMBED + c * LANES, LANES)])
                        for c in range(NCH)))
            return acc
        return lax.fori_loop(0, HALF // 8, body, acc)

    zeros = tuple(jnp.zeros((LANES,), jnp.float32) for _ in range(NCH))
    for u in range(NB - 1):
        fire(u, bufs[u], sems[u])

    def gbody(g, carry):
        acc = zeros
        for j in range(NB):
            u = NB * g + j
            drain(bufs[j], sems[j])
            if j == NB - 1:
                b = u // NB
                for c in range(NCH):
                    out_v[b, pl.ds(c * LANES, LANES)] = acc[c]
                acc = zeros
            jn = (j + NB - 1) % NB

            @pl.when(u + NB - 1 < UPW)
            def _():
                fire(u + NB - 1, bufs[jn], sems[jn])

        return carry

    lax.fori_loop(0, UPW // NB, gbody, 0)

    pltpu.sync_copy(out_v, out_hbm.at[pl.ds(base, BPW)])


_mesh = plsc.VectorSubcoreMesh(core_axis_name="c", subcore_axis_name="s")

_enc = functools.partial(
    pl.kernel, mesh=_mesh,
    compiler_params=pltpu.CompilerParams(use_tc_tiling_on_sc=False),
    out_type=jax.ShapeDtypeStruct((BATCH, EMBED), jnp.float32),
    scratch_types=[
        pltpu.VMEM((BPW * LP + 16,), jnp.int32),
        pltpu.VMEM((UPW, HALF), jnp.int32),
        pltpu.VMEM((LP, EMBED), jnp.float32),
        pltpu.VMEM((HALF, 2 * EMBED), jnp.float32),
        pltpu.VMEM((HALF, 2 * EMBED), jnp.float32),
        pltpu.VMEM((HALF, 2 * EMBED), jnp.float32),
        pltpu.VMEM((HALF, 2 * EMBED), jnp.float32),
        pltpu.VMEM((BPW, EMBED), jnp.float32),
        pltpu.SemaphoreType.DMA,
        pltpu.SemaphoreType.DMA,
        pltpu.SemaphoreType.DMA,
        pltpu.SemaphoreType.DMA,
    ],
)(_encoder)


@jax.jit
def kernel(x, table, f):
    xp = jnp.pad(x.astype(jnp.int32), ((0, 0), (0, LP - MAX_LEN))).reshape(-1)
    xs = jnp.right_shift(xp, 1).reshape(-1, HALF)
    fp = jnp.pad(f, ((0, LP - MAX_LEN), (0, 0)))
    t128 = table.reshape(VOCAB2, 2 * EMBED)
    return _enc(xp, xs, t128, fp)


# speedup vs baseline: 4.8974x; 4.8974x over previous
import functools

import jax
import jax.numpy as jnp
from jax import lax
from jax.experimental import pallas as pl
from jax.experimental.pallas import tpu as pltpu
from jax.experimental.pallas import tpu_sc as plsc

BATCH = 4096
MAX_LEN = 200
EMBED = 64
NC, NS, LANES = 2, 16, 16
NW = NC * NS
BPW = BATCH // NW
CA, CB = 104, 96
NCH = EMBED // LANES
NB = 4


def _encoder(x_hbm, tb_hbm, f_hbm, out_hbm,
             idx_v, f_v, rows0, rows1, rows2, rows3, out_v,
             sem0, sem1, sem2, sem3):
    bufs = (rows0, rows1, rows2, rows3)
    sems = (sem0, sem1, sem2, sem3)
    wid = lax.axis_index("s") * NC + lax.axis_index("c")
    base = wid * BPW

    pltpu.sync_copy(x_hbm.at[pl.ds(base, BPW)], idx_v)
    pltpu.sync_copy(f_hbm, f_v)

    def fire(b, rows, sem):
        pltpu.make_async_copy(
            tb_hbm.at[idx_v.at[b, pl.ds(0, CA)]],
            rows.at[pl.ds(0, CA)], sem).start()
        pltpu.make_async_copy(
            tb_hbm.at[idx_v.at[b, pl.ds(CA, CB)]],
            rows.at[pl.ds(CA, CB)], sem).start()

    def drain(rows, sem):
        pltpu.make_async_copy(
            tb_hbm.at[idx_v.at[0, pl.ds(0, CA)]],
            rows.at[pl.ds(0, CA)], sem).wait()
        pltpu.make_async_copy(
            tb_hbm.at[idx_v.at[0, pl.ds(CA, CB)]],
            rows.at[pl.ds(CA, CB)], sem).wait()

    def accumulate(b, rows):
        U = 4

        def body(i, acc):
            l0 = i * U
            for u in range(U):
                l = l0 + u
                for q in range(2):
                    v = rows[l, pl.ds(q * 2 * LANES, 2 * LANES)]
                    w = v.astype(jnp.float32)
                    a = acc[2 * q] + w[:LANES] \
                        * f_v[l, pl.ds(2 * q * LANES, LANES)]
                    c = acc[2 * q + 1] + w[LANES:] \
                        * f_v[l, pl.ds((2 * q + 1) * LANES, LANES)]
                    acc = (a, c, acc[2], acc[3]) if q == 0 \
                        else (acc[0], acc[1], a, c)
            return acc
        acc = lax.fori_loop(
            0, MAX_LEN // U, body,
            tuple(jnp.zeros((LANES,), jnp.float32) for _ in range(NCH)))
        for c in range(NCH):
            out_v[b, pl.ds(c * LANES, LANES)] = acc[c]

    for j in range(NB - 1):
        fire(j, bufs[j], sems[j])

    def gbody(g, carry):
        for j in range(NB):
            b = NB * g + j
            drain(bufs[j], sems[j])
            accumulate(b, bufs[j])
            jn = (j + NB - 1) % NB

            @pl.when(b + NB - 1 < BPW)
            def _():
                fire(b + NB - 1, bufs[jn], sems[jn])

        return carry

    lax.fori_loop(0, BPW // NB, gbody, 0)

    pltpu.sync_copy(out_v, out_hbm.at[pl.ds(base, BPW)])


_mesh = plsc.VectorSubcoreMesh(core_axis_name="c", subcore_axis_name="s")

_enc = functools.partial(
    pl.kernel, mesh=_mesh,
    compiler_params=pltpu.CompilerParams(use_tc_tiling_on_sc=False),
    out_type=jax.ShapeDtypeStruct((BATCH, EMBED), jnp.float32),
    scratch_types=[
        pltpu.VMEM((BPW, MAX_LEN), jnp.int32),
        pltpu.VMEM((MAX_LEN, EMBED), jnp.float32),
        pltpu.VMEM((MAX_LEN, EMBED), jnp.bfloat16),
        pltpu.VMEM((MAX_LEN, EMBED), jnp.bfloat16),
        pltpu.VMEM((MAX_LEN, EMBED), jnp.bfloat16),
        pltpu.VMEM((MAX_LEN, EMBED), jnp.bfloat16),
        pltpu.VMEM((BPW, EMBED), jnp.float32),
        pltpu.SemaphoreType.DMA,
        pltpu.SemaphoreType.DMA,
        pltpu.SemaphoreType.DMA,
        pltpu.SemaphoreType.DMA,
    ],
)(_encoder)


@jax.jit
def kernel(x, table, f):
    return _enc(x.astype(jnp.int32), table.astype(jnp.bfloat16), f)


# speedup vs baseline: 4.9023x; 1.0010x over previous
import functools

import jax
import jax.numpy as jnp
from jax import lax
from jax.experimental import pallas as pl
from jax.experimental.pallas import tpu as pltpu
from jax.experimental.pallas import tpu_sc as plsc

BATCH = 4096
MAX_LEN = 200
EMBED = 64
NC, NS, LANES = 2, 16, 16
NW = NC * NS
BPW = BATCH // NW
CA, CB = 104, 96
NCH = EMBED // LANES
NB = 4


def _encoder(x_hbm, tb_hbm, f_hbm, out_hbm,
             idx_v, f_v, rows0, rows1, rows2, rows3, out_v,
             sem0, sem1, sem2, sem3):
    bufs = (rows0, rows1, rows2, rows3)
    sems = (sem0, sem1, sem2, sem3)
    wid = lax.axis_index("s") * NC + lax.axis_index("c")
    base = wid * BPW

    pltpu.sync_copy(x_hbm.at[pl.ds(base, BPW)], idx_v)
    pltpu.sync_copy(f_hbm, f_v)

    def fire(b, rows, sem):
        pltpu.make_async_copy(
            tb_hbm.at[idx_v.at[b, pl.ds(0, CA)]],
            rows.at[pl.ds(0, CA)], sem).start()
        pltpu.make_async_copy(
            tb_hbm.at[idx_v.at[b, pl.ds(CA, CB)]],
            rows.at[pl.ds(CA, CB)], sem).start()

    def drain(rows, sem):
        pltpu.make_async_copy(
            tb_hbm.at[idx_v.at[0, pl.ds(0, CA)]],
            rows.at[pl.ds(0, CA)], sem).wait()
        pltpu.make_async_copy(
            tb_hbm.at[idx_v.at[0, pl.ds(CA, CB)]],
            rows.at[pl.ds(CA, CB)], sem).wait()

    def accumulate(b, rows):
        U = 4

        def body(i, acc):
            l0 = i * U
            for u in range(U):
                l = l0 + u
                for q in range(2):
                    v = rows[l, pl.ds(q * 2 * LANES, 2 * LANES)]
                    w = v.astype(jnp.float32)
                    a = acc[2 * q] + w[:LANES] \
                        * f_v[l, pl.ds(2 * q * LANES, LANES)]
                    c = acc[2 * q + 1] + w[LANES:] \
                        * f_v[l, pl.ds((2 * q + 1) * LANES, LANES)]
                    acc = (a, c, acc[2], acc[3]) if q == 0 \
                        else (acc[0], acc[1], a, c)
            return acc
        acc = lax.fori_loop(
            0, MAX_LEN // U, body,
            tuple(jnp.zeros((LANES,), jnp.float32) for _ in range(NCH)))
        for c in range(NCH):
            out_v[b, pl.ds(c * LANES, LANES)] = acc[c]

    for j in range(NB - 1):
        fire(j, bufs[j], sems[j])

    def gbody(g, carry):
        for j in range(NB):
            b = NB * g + j
            drain(bufs[j], sems[j])
            accumulate(b, bufs[j])
            jn = (j + NB - 1) % NB

            @pl.when(b + NB - 1 < BPW)
            def _():
                fire(b + NB - 1, bufs[jn], sems[jn])

        return carry

    lax.fori_loop(0, BPW // NB, gbody, 0)

    pltpu.sync_copy(out_v, out_hbm.at[pl.ds(base, BPW)])


_mesh = plsc.VectorSubcoreMesh(core_axis_name="c", subcore_axis_name="s")

_enc = functools.partial(
    pl.kernel, mesh=_mesh,
    compiler_params=pltpu.CompilerParams(use_tc_tiling_on_sc=False),
    out_type=jax.ShapeDtypeStruct((BATCH, EMBED), jnp.float32),
    scratch_types=[
        pltpu.VMEM((BPW, MAX_LEN), jnp.int32),
        pltpu.VMEM((MAX_LEN, EMBED), jnp.float32),
        pltpu.VMEM((MAX_LEN, EMBED), jnp.bfloat16),
        pltpu.VMEM((MAX_LEN, EMBED), jnp.bfloat16),
        pltpu.VMEM((MAX_LEN, EMBED), jnp.bfloat16),
        pltpu.VMEM((MAX_LEN, EMBED), jnp.bfloat16),
        pltpu.VMEM((BPW, EMBED), jnp.float32),
        pltpu.SemaphoreType.DMA,
        pltpu.SemaphoreType.DMA,
        pltpu.SemaphoreType.DMA,
        pltpu.SemaphoreType.DMA,
    ],
)(_encoder)


@jax.jit
def kernel(x, table, f):
    tb = lax.optimization_barrier(table.astype(jnp.bfloat16))
    return _enc(x.astype(jnp.int32), tb, f)


# speedup vs baseline: 6.0864x; 1.2415x over previous
import functools

import jax
import jax.numpy as jnp
from jax import lax
from jax.experimental import pallas as pl
from jax.experimental.pallas import tpu as pltpu
from jax.experimental.pallas import tpu_sc as plsc

BATCH = 4096
MAX_LEN = 200
EMBED = 64
NC, NS, LANES = 2, 16, 16
NW = NC * NS
BPW = BATCH // NW
CA, CB = 104, 96
NCH = EMBED // LANES
NB = 8
UPW = 2 * BPW


def _encoder(x_hbm, table_hbm, f_hbm, out_hbm,
             idx_v, f_v, b0, b1, b2, b3, b4, b5, b6, b7, out_v,
             s0, s1, s2, s3, s4, s5, s6, s7):
    bufs = (b0, b1, b2, b3, b4, b5, b6, b7)
    sems = (s0, s1, s2, s3, s4, s5, s6, s7)
    wid = lax.axis_index("s") * NC + lax.axis_index("c")
    base = wid * BPW

    pltpu.sync_copy(x_hbm.at[pl.ds(base, BPW)], idx_v)
    pltpu.sync_copy(f_hbm, f_v)

    def fire(u, h, rows, sem):
        b = u // 2
        if h == 0:
            src = table_hbm.at[idx_v.at[b, pl.ds(0, CA)]]
            dst = rows.at[pl.ds(0, CA)]
        else:
            src = table_hbm.at[idx_v.at[b, pl.ds(CA, CB)]]
            dst = rows.at[pl.ds(0, CB)]
        pltpu.make_async_copy(src, dst, sem).start()

    def drain(h, rows, sem):
        n = CA if h == 0 else CB
        pltpu.make_async_copy(
            table_hbm.at[idx_v.at[0, pl.ds(0, n)]],
            rows.at[pl.ds(0, n)], sem).wait()

    def accumulate(h, rows, acc):
        n, fb = (CA, 0) if h == 0 else (CB, CA)
        U = 8

        def body(i, acc):
            l0 = i * U
            for u in range(U):
                acc = tuple(
                    acc[c] + rows[l0 + u, pl.ds(c * LANES, LANES)]
                    * f_v[fb + l0 + u, pl.ds(c * LANES, LANES)]
                    for c in range(NCH))
            return acc
        return lax.fori_loop(0, n // U, body, acc)

    zeros = tuple(jnp.zeros((LANES,), jnp.float32) for _ in range(NCH))
    for u in range(NB - 1):
        fire(u, u % 2, bufs[u], sems[u])

    def gbody(g, carry):
        acc = zeros
        for j in range(NB):
            u = NB * g + j
            h = j % 2
            drain(h, bufs[j], sems[j])
            acc = accumulate(h, bufs[j], acc)
            if h:
                b = u // 2
                for c in range(NCH):
                    out_v[b, pl.ds(c * LANES, LANES)] = acc[c]
                acc = zeros
            jn = (j + NB - 1) % NB

            @pl.when(u + NB - 1 < UPW)
            def _():
                fire(u + NB - 1, jn % 2, bufs[jn], sems[jn])

        return carry

    lax.fori_loop(0, UPW // NB, gbody, 0)

    pltpu.sync_copy(out_v, out_hbm.at[pl.ds(base, BPW)])


_mesh = plsc.VectorSubcoreMesh(core_axis_name="c", subcore_axis_name="s")

_enc = functools.partial(
    pl.kernel, mesh=_mesh,
    compiler_params=pltpu.CompilerParams(use_tc_tiling_on_sc=False),
    out_type=jax.ShapeDtypeStruct((BATCH, EMBED), jnp.float32),
    scratch_types=[
        pltpu.VMEM((BPW, MAX_LEN), jnp.int32),
        pltpu.VMEM((MAX_LEN, EMBED), jnp.float32),
    ] + [pltpu.VMEM((CA, EMBED), jnp.float32)] * NB
      + [pltpu.VMEM((BPW, EMBED), jnp.float32)]
      + [pltpu.SemaphoreType.DMA] * NB,
)(_encoder)


@jax.jit
def kernel(x, table, f):
    return _enc(x.astype(jnp.int32), table, f)


# speedup vs baseline: 6.1114x; 1.0041x over previous
import functools

import jax
import jax.numpy as jnp
from jax import lax
from jax.experimental import pallas as pl
from jax.experimental.pallas import tpu as pltpu
from jax.experimental.pallas import tpu_sc as plsc

BATCH = 4096
MAX_LEN = 200
EMBED = 64
NC, NS, LANES = 2, 16, 16
NW = NC * NS
BPW = BATCH // NW
CA, CB = 104, 96
NCH = EMBED // LANES
NB = 4


def _encoder(x_hbm, table_hbm, f_hbm, out_hbm,
             idx_v, f_v, rows0, rows1, rows2, rows3, out_v,
             sem0, sem1, sem2, sem3):
    bufs = (rows0, rows1, rows2, rows3)
    sems = (sem0, sem1, sem2, sem3)
    wid = lax.axis_index("s") * NC + lax.axis_index("c")
    base = wid * BPW

    pltpu.sync_copy(x_hbm.at[pl.ds(base, BPW)], idx_v)
    pltpu.sync_copy(f_hbm, f_v)

    def fire(b, rows, sem):
        pltpu.make_async_copy(
            table_hbm.at[idx_v.at[b, pl.ds(0, CA)]],
            rows.at[pl.ds(0, CA)], sem).start()
        pltpu.make_async_copy(
            table_hbm.at[idx_v.at[b, pl.ds(CA, CB)]],
            rows.at[pl.ds(CA, CB)], sem).start()

    def drain(rows, sem):
        pltpu.make_async_copy(
            table_hbm.at[idx_v.at[0, pl.ds(0, CA)]],
            rows.at[pl.ds(0, CA)], sem).wait()
        pltpu.make_async_copy(
            table_hbm.at[idx_v.at[0, pl.ds(CA, CB)]],
            rows.at[pl.ds(CA, CB)], sem).wait()

    def accumulate(b, rows):
        U = 8

        def body(i, acc):
            l0 = i * U
            for u in range(U):
                acc = tuple(
                    acc[c] + rows[l0 + u, pl.ds(c * LANES, LANES)]
                    * f_v[l0 + u, pl.ds(c * LANES, LANES)]
                    for c in range(NCH))
            return acc
        acc = lax.fori_loop(
            0, MAX_LEN // U, body,
            tuple(jnp.zeros((LANES,), jnp.float32) for _ in range(NCH)))
        for c in range(NCH):
            out_v[b, pl.ds(c * LANES, LANES)] = acc[c]

    for j in range(NB - 1):
        fire(j, bufs[j], sems[j])

    def gbody(g, carry):
        for j in range(NB):
            b = NB * g + j
            drain(bufs[j], sems[j])
            accumulate(b, bufs[j])
            jn = (j + NB - 1) % NB

            @pl.when(b + NB - 1 < BPW)
            def _():
                fire(b + NB - 1, bufs[jn], sems[jn])

        return carry

    lax.fori_loop(0, BPW // NB, gbody, 0)

    pltpu.sync_copy(out_v, out_hbm.at[pl.ds(base, BPW)])


_mesh = plsc.VectorSubcoreMesh(core_axis_name="c", subcore_axis_name="s")

_enc = functools.partial(
    pl.kernel, mesh=_mesh,
    compiler_params=pltpu.CompilerParams(use_tc_tiling_on_sc=False),
    out_type=jax.ShapeDtypeStruct((BATCH, EMBED), jnp.float32),
    scratch_types=[
        pltpu.VMEM((BPW, MAX_LEN), jnp.int32),
        pltpu.VMEM((MAX_LEN, EMBED), jnp.float32),
        pltpu.VMEM((MAX_LEN, EMBED), jnp.float32),
        pltpu.VMEM((MAX_LEN, EMBED), jnp.float32),
        pltpu.VMEM((MAX_LEN, EMBED), jnp.float32),
        pltpu.VMEM((MAX_LEN, EMBED), jnp.float32),
        pltpu.VMEM((BPW, EMBED), jnp.float32),
        pltpu.SemaphoreType.DMA,
        pltpu.SemaphoreType.DMA,
        pltpu.SemaphoreType.DMA,
        pltpu.SemaphoreType.DMA,
    ],
)(_encoder)


@jax.jit
def kernel(x, table, f):
    return _enc(x.astype(jnp.int32), table, f)
